# Initial kernel scaffold; baseline (speedup 1.0000x reference)
import jax
import jax.numpy as jnp
from jax.experimental import pallas as pl


def kernel(z_e, embed_weight):
    raise NotImplementedError("write your pallas kernel here")


# speedup vs baseline: 1.0788x; 1.0788x over previous
import functools

import jax
import jax.numpy as jnp
from jax import lax
from jax.experimental import pallas as pl
from jax.experimental.pallas import tpu as pltpu
from jax.experimental.pallas import tpu_sc as plsc

_NUM_EMBED = 8192
_EMBED_DIM = 256
_BETA = 0.25
_KLD_SCALE = 10.0

_M_TILE = 256
_N_CHUNK = 2048

_NW = 32
_G_CH = 128
_N_GCH = 8

_L_TILE = 512


def _argmin_body(x_ref, xn_ref, e_ref, en_ref, idx_ref):
    x = x_ref[...]
    xn = xn_ref[...]
    run_m = None
    run_i = None
    for c in range(_NUM_EMBED // _N_CHUNK):
        lo = c * _N_CHUNK
        e = e_ref[lo:lo + _N_CHUNK, :]
        en = en_ref[:, lo:lo + _N_CHUNK]
        mm = lax.dot_general(x, e, (((1,), (1,)), ((), ())),
                             preferred_element_type=jnp.float32)
        dist = (xn - 2.0 * mm) + en
        m = jnp.min(dist, axis=1, keepdims=True)
        ii = lax.broadcasted_iota(jnp.int32, dist.shape, 1) + lo
        cand = jnp.min(jnp.where(dist == m, ii, jnp.int32(2**30)),
                       axis=1, keepdims=True)
        if run_m is None:
            run_m, run_i = m, cand
        else:
            take = m < run_m
            run_i = jnp.where(take, cand, run_i)
            run_m = jnp.where(take, m, run_m)
    idx_ref[...] = run_i


def _argmin_call(flatten, xn, embed_weight, en):
    n_rows = flatten.shape[0]
    grid = (n_rows // _M_TILE,)
    return pl.pallas_call(
        _argmin_body,
        grid=grid,
        in_specs=[
            pl.BlockSpec((_M_TILE, _EMBED_DIM), lambda i: (i, 0)),
            pl.BlockSpec((_M_TILE, 1), lambda i: (i, 0)),
            pl.BlockSpec((_NUM_EMBED, _EMBED_DIM), lambda i: (0, 0)),
            pl.BlockSpec((1, _NUM_EMBED), lambda i: (0, 0)),
        ],
        out_specs=pl.BlockSpec((_M_TILE, 1), lambda i: (i, 0)),
        out_shape=jax.ShapeDtypeStruct((n_rows, 1), jnp.int32),
    )(flatten, xn, embed_weight, en)


def _sc_gather_body(table_hbm, idx_hbm, out_hbm, idx_v, rows_v, sem):
    cid = lax.axis_index("c")
    sid = lax.axis_index("s")
    wid = sid * 2 + cid
    base = wid * (_N_GCH * _G_CH)
    pltpu.sync_copy(idx_hbm.at[wid], idx_v)
    handles = [None, None]
    handles[0] = pltpu.async_copy(table_hbm.at[idx_v.at[0]], rows_v.at[0], sem)
    for k in range(_N_GCH):
        if k + 1 < _N_GCH:
            handles[(k + 1) % 2] = pltpu.async_copy(
                table_hbm.at[idx_v.at[k + 1]], rows_v.at[(k + 1) % 2], sem)
        handles[k % 2].wait()
        pltpu.sync_copy(rows_v.at[k % 2],
                        out_hbm.at[pl.ds(base + k * _G_CH, _G_CH)])


def _gather_call(embed_weight, idx3):
    n_rows = _NW * _N_GCH * _G_CH
    mesh = plsc.VectorSubcoreMesh(core_axis_name="c", subcore_axis_name="s")
    run = functools.partial(
        pl.kernel,
        mesh=mesh,
        out_type=jax.ShapeDtypeStruct((n_rows, _EMBED_DIM), jnp.float32),
        scratch_types=[
            pltpu.VMEM((_N_GCH, _G_CH), jnp.int32),
            pltpu.VMEM((2, _G_CH, _EMBED_DIM), jnp.float32),
            pltpu.SemaphoreType.DMA,
        ],
    )(_sc_gather_body)
    return run(embed_weight, idx3)


def _finish_body(zq_ref, ze_ref, out_ref, acc_ref):
    b = pl.program_id(0)
    l = pl.program_id(1)
    q = zq_ref[0]
    e = ze_ref[0]
    d = q - e
    st = e + d
    out_ref[0] = st.T

    @pl.when((b == 0) & (l == 0))
    def _():
        acc_ref[...] = jnp.zeros_like(acc_ref)

    acc_ref[...] += jnp.sum(d * d).reshape(1, 1)


def _finish_call(z_q, z_e):
    b, l, d = z_e.shape
    grid = (b, l // _L_TILE)
    return pl.pallas_call(
        _finish_body,
        grid=grid,
        in_specs=[
            pl.BlockSpec((1, _L_TILE, d), lambda i, j: (i, j, 0)),
            pl.BlockSpec((1, _L_TILE, d), lambda i, j: (i, j, 0)),
        ],
        out_specs=[
            pl.BlockSpec((1, d, _L_TILE), lambda i, j: (i, 0, j)),
            pl.BlockSpec((1, 1), lambda i, j: (0, 0)),
        ],
        out_shape=[
            jax.ShapeDtypeStruct((b, d, l), jnp.float32),
            jax.ShapeDtypeStruct((1, 1), jnp.float32),
        ],
    )(z_q, z_e)


def kernel(z_e, embed_weight):
    b, l, d = z_e.shape
    flatten = z_e.reshape(-1, d)
    xn = (flatten ** 2).sum(axis=1, keepdims=True)
    en = (embed_weight ** 2).sum(axis=1)[None, :]

    idx = _argmin_call(flatten, xn, embed_weight, en).reshape(-1)
    z_q = _gather_call(embed_weight, idx.reshape(_NW, _N_GCH, _G_CH))
    z_q_out, acc = _finish_call(z_q.reshape(b, l, d), z_e)

    m = acc[0, 0] / (b * l * d)
    diff = (m + _BETA * m) * _KLD_SCALE
    return (z_q_out, diff, idx.reshape(b, l))
